# baseline (device time: 64686 ns/iter reference)
import jax
import jax.numpy as jnp
from jax import lax
from jax.experimental import pallas as pl
from jax.experimental.pallas import tpu as pltpu

CHUNK_ROWS = [128] * 7 + [64, 32, 16, 8, 8]
N_CHUNK = len(CHUNK_ROWS)
_MESH = pl.DeviceIdType.MESH


def kernel(x, pi):
    _, rows_total, _ = x.shape
    half = rows_total // 2
    assert sum(CHUNK_ROWS) == half
    offs = [sum(CHUNK_ROWS[:c]) for c in range(N_CHUNK)]

    def body(x_ref, pi_ref, out_ref, xs_sems, xr_sems, ys_sems, yr_sems,
             local_sem):
        my_x = lax.axis_index("x")
        my_y = lax.axis_index("y")
        tgt = pi_ref[my_x]
        swap = tgt != my_x

        @pl.when(swap)
        def _():
            x_nbr = (tgt, my_y)
            y_nbr = (my_x, 1 - my_y)

            barrier = pltpu.get_barrier_semaphore()
            for nbr in (x_nbr, y_nbr):
                pl.semaphore_signal(barrier, inc=1, device_id=nbr,
                                    device_id_type=_MESH)
            pl.semaphore_wait(barrier, 2)

            my_base = my_y * half
            other_base = (1 - my_y) * half

            x_rdmas = []
            for c in range(N_CHUNK):
                rows = pl.ds(my_base + offs[c], CHUNK_ROWS[c])
                rdma = pltpu.make_async_remote_copy(
                    src_ref=x_ref.at[:, rows, :],
                    dst_ref=out_ref.at[:, rows, :],
                    send_sem=xs_sems.at[c],
                    recv_sem=xr_sems.at[c],
                    device_id=x_nbr,
                    device_id_type=_MESH,
                )
                rdma.start()
                x_rdmas.append(rdma)

            y_sends = []
            for c in range(N_CHUNK):
                rows = pl.ds(my_base + offs[c], CHUNK_ROWS[c])
                x_rdmas[c].wait_recv()
                fwd = pltpu.make_async_remote_copy(
                    src_ref=out_ref.at[:, rows, :],
                    dst_ref=out_ref.at[:, rows, :],
                    send_sem=ys_sems.at[c],
                    recv_sem=yr_sems.at[c],
                    device_id=y_nbr,
                    device_id_type=_MESH,
                )
                fwd.start()
                y_sends.append(fwd)

            for c in range(N_CHUNK):
                orows = pl.ds(other_base + offs[c], CHUNK_ROWS[c])
                recv = pltpu.make_async_remote_copy(
                    src_ref=out_ref.at[:, orows, :],
                    dst_ref=out_ref.at[:, orows, :],
                    send_sem=ys_sems.at[c],
                    recv_sem=yr_sems.at[c],
                    device_id=y_nbr,
                    device_id_type=_MESH,
                )
                recv.wait_recv()
                y_sends[c].wait_send()
                x_rdmas[c].wait_send()

        @pl.when(jnp.logical_not(swap))
        def _():
            copy = pltpu.make_async_copy(x_ref, out_ref, local_sem)
            copy.start()
            copy.wait()

    return pl.pallas_call(
        body,
        out_shape=jax.ShapeDtypeStruct(x.shape, x.dtype),
        in_specs=[
            pl.BlockSpec(memory_space=pltpu.MemorySpace.HBM),
            pl.BlockSpec(memory_space=pltpu.SMEM),
        ],
        out_specs=pl.BlockSpec(memory_space=pltpu.MemorySpace.HBM),
        scratch_shapes=[
            pltpu.SemaphoreType.DMA((N_CHUNK,)),
            pltpu.SemaphoreType.DMA((N_CHUNK,)),
            pltpu.SemaphoreType.DMA((N_CHUNK,)),
            pltpu.SemaphoreType.DMA((N_CHUNK,)),
            pltpu.SemaphoreType.DMA,
        ],
        compiler_params=pltpu.CompilerParams(collective_id=0),
    )(x, pi)


# device time: 64067 ns/iter; 1.0097x vs baseline; 1.0097x over previous
import jax
import jax.numpy as jnp
from jax import lax
from jax.experimental import pallas as pl
from jax.experimental.pallas import tpu as pltpu

CHUNK_ROWS = [8, 16, 40] + [112] * 8 + [40, 16, 8]
N_CHUNK = len(CHUNK_ROWS)
_MESH = pl.DeviceIdType.MESH


def kernel(x, pi):
    _, rows_total, _ = x.shape
    half = rows_total // 2
    assert sum(CHUNK_ROWS) == half
    offs = [sum(CHUNK_ROWS[:c]) for c in range(N_CHUNK)]

    def body(x_ref, pi_ref, out_ref, xs_sems, xr_sems, ys_sems, yr_sems,
             local_sem):
        my_x = lax.axis_index("x")
        my_y = lax.axis_index("y")
        tgt = pi_ref[my_x]
        swap = tgt != my_x

        @pl.when(swap)
        def _():
            x_nbr = (tgt, my_y)
            y_nbr = (my_x, 1 - my_y)

            barrier = pltpu.get_barrier_semaphore()
            for nbr in (x_nbr, y_nbr):
                pl.semaphore_signal(barrier, inc=1, device_id=nbr,
                                    device_id_type=_MESH)
            pl.semaphore_wait(barrier, 2)

            my_base = my_y * half
            other_base = (1 - my_y) * half

            x_rdmas = []
            for c in range(N_CHUNK):
                rows = pl.ds(my_base + offs[c], CHUNK_ROWS[c])
                rdma = pltpu.make_async_remote_copy(
                    src_ref=x_ref.at[:, rows, :],
                    dst_ref=out_ref.at[:, rows, :],
                    send_sem=xs_sems.at[c],
                    recv_sem=xr_sems.at[c],
                    device_id=x_nbr,
                    device_id_type=_MESH,
                )
                rdma.start()
                x_rdmas.append(rdma)

            y_sends = []
            for c in range(N_CHUNK):
                rows = pl.ds(my_base + offs[c], CHUNK_ROWS[c])
                x_rdmas[c].wait_recv()
                fwd = pltpu.make_async_remote_copy(
                    src_ref=out_ref.at[:, rows, :],
                    dst_ref=out_ref.at[:, rows, :],
                    send_sem=ys_sems.at[c],
                    recv_sem=yr_sems.at[c],
                    device_id=y_nbr,
                    device_id_type=_MESH,
                )
                fwd.start()
                y_sends.append(fwd)

            for c in range(N_CHUNK):
                orows = pl.ds(other_base + offs[c], CHUNK_ROWS[c])
                recv = pltpu.make_async_remote_copy(
                    src_ref=out_ref.at[:, orows, :],
                    dst_ref=out_ref.at[:, orows, :],
                    send_sem=ys_sems.at[c],
                    recv_sem=yr_sems.at[c],
                    device_id=y_nbr,
                    device_id_type=_MESH,
                )
                recv.wait_recv()
                y_sends[c].wait_send()
                x_rdmas[c].wait_send()

        @pl.when(jnp.logical_not(swap))
        def _():
            copy = pltpu.make_async_copy(x_ref, out_ref, local_sem)
            copy.start()
            copy.wait()

    return pl.pallas_call(
        body,
        out_shape=jax.ShapeDtypeStruct(x.shape, x.dtype),
        in_specs=[
            pl.BlockSpec(memory_space=pltpu.MemorySpace.HBM),
            pl.BlockSpec(memory_space=pltpu.SMEM),
        ],
        out_specs=pl.BlockSpec(memory_space=pltpu.MemorySpace.HBM),
        scratch_shapes=[
            pltpu.SemaphoreType.DMA((N_CHUNK,)),
            pltpu.SemaphoreType.DMA((N_CHUNK,)),
            pltpu.SemaphoreType.DMA((N_CHUNK,)),
            pltpu.SemaphoreType.DMA((N_CHUNK,)),
            pltpu.SemaphoreType.DMA,
        ],
        compiler_params=pltpu.CompilerParams(collective_id=0),
    )(x, pi)


# device time: 61648 ns/iter; 1.0493x vs baseline; 1.0392x over previous
import jax
import jax.numpy as jnp
from jax import lax
from jax.experimental import pallas as pl
from jax.experimental.pallas import tpu as pltpu

CHUNK_ROWS = [16] * 64
N_CHUNK = len(CHUNK_ROWS)
_MESH = pl.DeviceIdType.MESH


def kernel(x, pi):
    _, rows_total, _ = x.shape
    half = rows_total // 2
    assert sum(CHUNK_ROWS) == half
    offs = [sum(CHUNK_ROWS[:c]) for c in range(N_CHUNK)]

    def body(x_ref, pi_ref, out_ref, xs_sems, xr_sems, ys_sems, yr_sems,
             local_sem):
        my_x = lax.axis_index("x")
        my_y = lax.axis_index("y")
        tgt = pi_ref[my_x]
        swap = tgt != my_x

        @pl.when(swap)
        def _():
            x_nbr = (tgt, my_y)
            y_nbr = (my_x, 1 - my_y)

            barrier = pltpu.get_barrier_semaphore()
            for nbr in (x_nbr, y_nbr):
                pl.semaphore_signal(barrier, inc=1, device_id=nbr,
                                    device_id_type=_MESH)
            pl.semaphore_wait(barrier, 2)

            my_base = my_y * half
            other_base = (1 - my_y) * half

            x_rdmas = []
            for c in range(N_CHUNK):
                rows = pl.ds(my_base + offs[c], CHUNK_ROWS[c])
                rdma = pltpu.make_async_remote_copy(
                    src_ref=x_ref.at[:, rows, :],
                    dst_ref=out_ref.at[:, rows, :],
                    send_sem=xs_sems.at[c],
                    recv_sem=xr_sems.at[c],
                    device_id=x_nbr,
                    device_id_type=_MESH,
                )
                rdma.start()
                x_rdmas.append(rdma)

            y_sends = []
            for c in range(N_CHUNK):
                rows = pl.ds(my_base + offs[c], CHUNK_ROWS[c])
                x_rdmas[c].wait_recv()
                fwd = pltpu.make_async_remote_copy(
                    src_ref=out_ref.at[:, rows, :],
                    dst_ref=out_ref.at[:, rows, :],
                    send_sem=ys_sems.at[c],
                    recv_sem=yr_sems.at[c],
                    device_id=y_nbr,
                    device_id_type=_MESH,
                )
                fwd.start()
                y_sends.append(fwd)

            for c in range(N_CHUNK):
                orows = pl.ds(other_base + offs[c], CHUNK_ROWS[c])
                recv = pltpu.make_async_remote_copy(
                    src_ref=out_ref.at[:, orows, :],
                    dst_ref=out_ref.at[:, orows, :],
                    send_sem=ys_sems.at[c],
                    recv_sem=yr_sems.at[c],
                    device_id=y_nbr,
                    device_id_type=_MESH,
                )
                recv.wait_recv()
                y_sends[c].wait_send()
                x_rdmas[c].wait_send()

        @pl.when(jnp.logical_not(swap))
        def _():
            copy = pltpu.make_async_copy(x_ref, out_ref, local_sem)
            copy.start()
            copy.wait()

    return pl.pallas_call(
        body,
        out_shape=jax.ShapeDtypeStruct(x.shape, x.dtype),
        in_specs=[
            pl.BlockSpec(memory_space=pltpu.MemorySpace.HBM),
            pl.BlockSpec(memory_space=pltpu.SMEM),
        ],
        out_specs=pl.BlockSpec(memory_space=pltpu.MemorySpace.HBM),
        scratch_shapes=[
            pltpu.SemaphoreType.DMA((N_CHUNK,)),
            pltpu.SemaphoreType.DMA((N_CHUNK,)),
            pltpu.SemaphoreType.DMA((N_CHUNK,)),
            pltpu.SemaphoreType.DMA((N_CHUNK,)),
            pltpu.SemaphoreType.DMA,
        ],
        compiler_params=pltpu.CompilerParams(collective_id=0),
    )(x, pi)


# device time: 60905 ns/iter; 1.0621x vs baseline; 1.0122x over previous
import jax
import jax.numpy as jnp
from jax import lax
from jax.experimental import pallas as pl
from jax.experimental.pallas import tpu as pltpu

CHUNK_ROWS = [32] * 32
N_CHUNK = len(CHUNK_ROWS)
_MESH = pl.DeviceIdType.MESH


def kernel(x, pi):
    _, rows_total, _ = x.shape
    half = rows_total // 2
    assert sum(CHUNK_ROWS) == half
    offs = [sum(CHUNK_ROWS[:c]) for c in range(N_CHUNK)]

    def body(x_ref, pi_ref, out_ref, xs_sems, xr_sems, ys_sems, yr_sems,
             local_sem):
        my_x = lax.axis_index("x")
        my_y = lax.axis_index("y")
        tgt = pi_ref[my_x]
        swap = tgt != my_x

        @pl.when(swap)
        def _():
            x_nbr = (tgt, my_y)
            y_nbr = (my_x, 1 - my_y)

            barrier = pltpu.get_barrier_semaphore()
            for nbr in (x_nbr, y_nbr):
                pl.semaphore_signal(barrier, inc=1, device_id=nbr,
                                    device_id_type=_MESH)
            pl.semaphore_wait(barrier, 2)

            my_base = my_y * half
            other_base = (1 - my_y) * half

            x_rdmas = []
            for c in range(N_CHUNK):
                rows = pl.ds(my_base + offs[c], CHUNK_ROWS[c])
                rdma = pltpu.make_async_remote_copy(
                    src_ref=x_ref.at[:, rows, :],
                    dst_ref=out_ref.at[:, rows, :],
                    send_sem=xs_sems.at[c],
                    recv_sem=xr_sems.at[c],
                    device_id=x_nbr,
                    device_id_type=_MESH,
                )
                rdma.start()
                x_rdmas.append(rdma)

            y_sends = []
            for c in range(N_CHUNK):
                rows = pl.ds(my_base + offs[c], CHUNK_ROWS[c])
                x_rdmas[c].wait_recv()
                fwd = pltpu.make_async_remote_copy(
                    src_ref=out_ref.at[:, rows, :],
                    dst_ref=out_ref.at[:, rows, :],
                    send_sem=ys_sems.at[c],
                    recv_sem=yr_sems.at[c],
                    device_id=y_nbr,
                    device_id_type=_MESH,
                )
                fwd.start()
                y_sends.append(fwd)

            for c in range(N_CHUNK):
                orows = pl.ds(other_base + offs[c], CHUNK_ROWS[c])
                recv = pltpu.make_async_remote_copy(
                    src_ref=out_ref.at[:, orows, :],
                    dst_ref=out_ref.at[:, orows, :],
                    send_sem=ys_sems.at[c],
                    recv_sem=yr_sems.at[c],
                    device_id=y_nbr,
                    device_id_type=_MESH,
                )
                recv.wait_recv()
                y_sends[c].wait_send()
                x_rdmas[c].wait_send()

        @pl.when(jnp.logical_not(swap))
        def _():
            copy = pltpu.make_async_copy(x_ref, out_ref, local_sem)
            copy.start()
            copy.wait()

    return pl.pallas_call(
        body,
        out_shape=jax.ShapeDtypeStruct(x.shape, x.dtype),
        in_specs=[
            pl.BlockSpec(memory_space=pltpu.MemorySpace.HBM),
            pl.BlockSpec(memory_space=pltpu.SMEM),
        ],
        out_specs=pl.BlockSpec(memory_space=pltpu.MemorySpace.HBM),
        scratch_shapes=[
            pltpu.SemaphoreType.DMA((N_CHUNK,)),
            pltpu.SemaphoreType.DMA((N_CHUNK,)),
            pltpu.SemaphoreType.DMA((N_CHUNK,)),
            pltpu.SemaphoreType.DMA((N_CHUNK,)),
            pltpu.SemaphoreType.DMA,
        ],
        compiler_params=pltpu.CompilerParams(collective_id=0),
    )(x, pi)
